# Initial kernel scaffold; baseline (speedup 1.0000x reference)
import jax
import jax.numpy as jnp
from jax.experimental import pallas as pl


def kernel(atom_list, bond_list, atom_degree_list, bond_degree_list, atom_mask, atom_fc_W, atom_fc_b, neighbor_fc_W, neighbor_fc_b, align_W, align_b, attend_W, attend_b, gru_wih, gru_bih, gru_whh, gru_bhh, mol_align_W, mol_align_b, mol_attend_W, mol_attend_b, molgru_wih, molgru_bih, molgru_whh, molgru_bhh, out_W, out_b):
    raise NotImplementedError("write your pallas kernel here")


# speedup vs baseline: 55.1825x; 55.1825x over previous
import jax
import jax.numpy as jnp
from jax.experimental import pallas as pl
from jax.experimental.pallas import tpu as pltpu

MB = 32
L = 128
NB = 6
FP = 64
T_STEPS = 2


def _dotT(x, w):
    return jax.lax.dot_general(x, w, (((1,), (1,)), ((), ())),
                               preferred_element_type=jnp.float32)


def _dot(x, w):
    return jax.lax.dot_general(x, w, (((1,), (0,)), ((), ())),
                               preferred_element_type=jnp.float32)


def _leaky(x):
    return jnp.where(x >= 0, x, 0.01 * x)


def _elu(x):
    return jnp.where(x > 0, x, jnp.exp(x) - 1.0)


def _softmax_sub(x):
    m = jnp.max(x, axis=1, keepdims=True)
    e = jnp.exp(x - m)
    return e / jnp.sum(e, axis=1, keepdims=True)


def _softmax_lanes(x):
    m = jnp.max(x, axis=-1, keepdims=True)
    e = jnp.exp(x - m)
    return e / jnp.sum(e, axis=-1, keepdims=True)


def _gru(x, h, wrzx, wrzh, brz, win, bin_, whn, bhn):
    rz = jax.nn.sigmoid(_dot(x, wrzx) + _dot(h, wrzh) + brz)
    r = rz[:, :FP]
    z = rz[:, FP:]
    hn = _dotT(h, whn) + bhn
    inn = _dotT(x, win) + bin_
    n = jnp.tanh(inn + r * hn)
    return (1.0 - z) * n + z * h


def _gather_t(valT, idx4):
    rep = jnp.broadcast_to(valT[:, None, :, :], (MB, NB, FP, L))
    return jnp.take_along_axis(rep, idx4, axis=3, mode="promise_in_bounds")


def _body(a_ref, b_ref, adt_ref, bdt_ref, m_ref,
          afcW, afcb, nWa, nWb, nfcb,
          alw1, alw2, alb, atW, atbT,
          gwrzx, gwrzh, gbrz, gwin, gbin, gwhn, gbhn,
          malw1, malw2, malb, matW, matb,
          mgwrzx, mgwrzh, mgbrz, mgwin, mgbin, mgwhn, mgbhn, outW, outb,
          afv, aawt, mfv, mfu, maw, pred):
    a3 = a_ref[...]
    b3 = b_ref[...]
    adt = adt_ref[...]
    bdt = bdt_ref[...]
    feat = a3.shape[-1]
    bond = b3.shape[-1]

    a2 = a3.reshape(MB * L, feat)
    b2 = b3.reshape(MB * L, bond)
    pre2 = _dotT(a2, afcW[...]) + afcb[...]
    afv[0] = pre2.reshape(MB, L, FP)
    atomf = _leaky(pre2)
    atomf3 = atomf.reshape(MB, L, FP)

    is_pad = adt == (L - 1)
    amask = jnp.where(is_pad, 0.0, 1.0).astype(jnp.float32)
    smask = jnp.where(is_pad, -9e8, 0.0).astype(jnp.float32)

    idxA4 = jnp.broadcast_to(adt[:, :, None, :], (MB, NB, FP, L))
    idxB4 = jnp.broadcast_to(bdt[:, :, None, :], (MB, NB, FP, L))

    alw1v = alw1[...]
    alw2v = alw2[...]
    atWv = atW[...]
    atbTv = atbT[...]

    nWa_v = nWa[...]
    nWb_v = nWb[...]
    paT = jnp.stack([_dotT(nWa_v, a3[m]) for m in range(MB)])
    pbT = jnp.stack([_dotT(nWb_v, b3[m]) for m in range(MB)])
    nbfT4 = _leaky(_gather_t(paT, idxA4) + _gather_t(pbT, idxB4)
                   + nfcb[...].reshape(1, 1, FP, 1))
    s1 = jnp.stack([_dotT(alw1v[0], atomf3[m]) for m in range(MB)])
    s2 = jnp.concatenate(
        [jnp.concatenate([_dot(alw2v[0], nbfT4[m, nb]) for nb in range(NB)],
                         axis=0)[None] for m in range(MB)], axis=0)
    score = _leaky(s1 + s2 + alb[0, 0, 0]) + smask
    w = _softmax_sub(score) * amask
    aawt[0] = w
    ctx_rows = []
    for m in range(MB):
        acc = None
        for nb in range(NB):
            nftT = _dot(atWv[0], nbfT4[m, nb]) + atbTv[0]
            term = nftT * w[m, nb:nb + 1, :]
            acc = term if acc is None else acc + term
        ctx_rows.append(jnp.swapaxes(_elu(acc), 0, 1))
    ctx2 = jnp.concatenate(ctx_rows, axis=0)
    h = _gru(ctx2, atomf, gwrzx[0], gwrzh[0], gbrz[0], gwin[0], gbin[0],
             gwhn[0], gbhn[0])
    act = jnp.maximum(h, 0.0)
    afv[1] = act.reshape(MB, L, FP)

    n_radius = alw1v.shape[0]
    for r in range(1, n_radius):
        act3 = act.reshape(MB, L, FP)
        attprojT = jnp.stack([_dotT(atWv[r], act3[m]) for m in range(MB)])
        s1 = jnp.stack([_dotT(alw1v[r], act3[m]) for m in range(MB)])
        p2 = jnp.stack([_dotT(alw2v[r], act3[m]) for m in range(MB)])
        p2rep = jnp.broadcast_to(p2, (MB, NB, L))
        s2 = jnp.take_along_axis(p2rep, adt, axis=2, mode="promise_in_bounds")
        score = _leaky(s1 + s2 + alb[r, 0, 0]) + smask
        w = _softmax_sub(score) * amask
        aawt[r] = w
        gat4 = _gather_t(attprojT, idxA4)
        ctxT = jnp.sum(gat4 * w[:, :, None, :], axis=1)
        wsum = jnp.sum(w, axis=1)
        ctxT = ctxT + atbTv[r][None] * wsum[:, None, :]
        ctxT = _elu(ctxT)
        ctx2 = jnp.concatenate(
            [jnp.swapaxes(ctxT[m], 0, 1) for m in range(MB)], axis=0)
        h = _gru(ctx2, h, gwrzx[r], gwrzh[r], gbrz[r], gwin[r], gbin[r],
                 gwhn[r], gbhn[r])
        act = jnp.maximum(h, 0.0)
        afv[r + 1] = act.reshape(MB, L, FP)

    mask2 = m_ref[...]
    h3 = h.reshape(MB, L, FP)
    act3 = act.reshape(MB, L, FP)
    mfu[0] = jnp.concatenate(
        [_dot(mask2[m:m + 1, :], h3[m]) for m in range(MB)], axis=0)
    molf = jnp.concatenate(
        [_dot(mask2[m:m + 1, :], act3[m]) for m in range(MB)], axis=0)
    mfv[0] = molf
    actmol = jnp.maximum(molf, 0.0)
    msmask = jnp.where(mask2 == 0.0, -9e8, 0.0).astype(jnp.float32)

    aft = (_dotT(act, matW[...]) + matb[...]).reshape(MB, L, FP)
    malw1v = malw1[...]
    malw2v = malw2[...]
    s2m = jnp.sum(act3 * malw2v[None], axis=-1)
    for t in range(T_STEPS):
        s1m = _dotT(actmol, malw1v)
        score = _leaky(s1m + s2m + malb[0, 0]) + msmask
        w = _softmax_lanes(score) * mask2
        maw[t] = w
        mctx = _elu(jnp.concatenate(
            [_dot(w[m:m + 1, :], aft[m]) for m in range(MB)], axis=0))
        molf = _gru(mctx, molf, mgwrzx[...], mgwrzh[...], mgbrz[...],
                    mgwin[...], mgbin[...], mgwhn[...], mgbhn[...])
        mfu[t + 1] = molf
        actmol = jnp.maximum(molf, 0.0)
        mfv[t + 1] = actmol
    pred[...] = jnp.sum(molf * outW[...], axis=-1, keepdims=True) + outb[0, 0]


def _split_gru(wih, bih, whh, bhh):
    wih_r, wih_z, wih_n = (wih[..., i * FP:(i + 1) * FP, :] for i in range(3))
    whh_r, whh_z, whh_n = (whh[..., i * FP:(i + 1) * FP, :] for i in range(3))
    bih_r, bih_z, bih_n = (bih[..., i * FP:(i + 1) * FP] for i in range(3))
    bhh_r, bhh_z, bhh_n = (bhh[..., i * FP:(i + 1) * FP] for i in range(3))
    swap = lambda w: jnp.swapaxes(w, -1, -2)
    top = jnp.concatenate([swap(wih_r), swap(wih_z)], axis=-1)
    bot = jnp.concatenate([swap(whh_r), swap(whh_z)], axis=-1)
    brz = jnp.concatenate([bih_r + bhh_r, bih_z + bhh_z], axis=-1)
    return (top, bot, brz[..., None, :], wih_n, bih_n[..., None, :],
            whh_n, bhh_n[..., None, :])


def kernel(atom_list, bond_list, atom_degree_list, bond_degree_list, atom_mask,
           atom_fc_W, atom_fc_b, neighbor_fc_W, neighbor_fc_b,
           align_W, align_b, attend_W, attend_b,
           gru_wih, gru_bih, gru_whh, gru_bhh,
           mol_align_W, mol_align_b, mol_attend_W, mol_attend_b,
           molgru_wih, molgru_bih, molgru_whh, molgru_bhh, out_W, out_b):
    B, Lm, feat = atom_list.shape
    bond = bond_list.shape[-1]
    n_radius = align_W.shape[0]
    grid = B // MB

    adt = jnp.swapaxes(atom_degree_list.astype(jnp.int32), 1, 2)
    bdt = jnp.swapaxes(bond_degree_list.astype(jnp.int32), 1, 2)

    afcb = atom_fc_b.reshape(1, FP)
    nWa = neighbor_fc_W[:, :feat]
    nWb = neighbor_fc_W[:, feat:]
    nfcb = neighbor_fc_b.reshape(1, FP)
    alw1 = align_W[:, :, :FP]
    alw2 = align_W[:, :, FP:]
    alb = align_b.reshape(n_radius, 1, 1)
    atbT = attend_b[..., None]
    gwrzx, gwrzh, gbrz, gwin, gbin, gwhn, gbhn = _split_gru(
        gru_wih, gru_bih, gru_whh, gru_bhh)
    malw1 = mol_align_W[:, :FP]
    malw2 = mol_align_W[:, FP:]
    malb = mol_align_b.reshape(1, 1)
    matb = mol_attend_b.reshape(1, FP)
    mgwrzx, mgwrzh, mgbrz, mgwin, mgbin, mgwhn, mgbhn = _split_gru(
        molgru_wih, molgru_bih, molgru_whh, molgru_bhh)
    outb = out_b.reshape(1, 1)

    def full(arr):
        nd = arr.ndim
        return pl.BlockSpec(arr.shape, lambda i, _n=nd: (0,) * _n)

    tensor_specs = [
        pl.BlockSpec((MB, Lm, feat), lambda i: (i, 0, 0)),
        pl.BlockSpec((MB, Lm, bond), lambda i: (i, 0, 0)),
        pl.BlockSpec((MB, NB, Lm), lambda i: (i, 0, 0)),
        pl.BlockSpec((MB, NB, Lm), lambda i: (i, 0, 0)),
        pl.BlockSpec((MB, Lm), lambda i: (i, 0)),
    ]
    weight_args = (atom_fc_W, afcb, nWa, nWb, nfcb,
                   alw1, alw2, alb, attend_W, atbT,
                   gwrzx, gwrzh, gbrz, gwin, gbin, gwhn, gbhn,
                   malw1, malw2, malb, mol_attend_W, matb,
                   mgwrzx, mgwrzh, mgbrz, mgwin, mgbin, mgwhn, mgbhn,
                   out_W, outb)
    weight_specs = [full(wa) for wa in weight_args]

    out_shapes = (
        jax.ShapeDtypeStruct((n_radius + 1, B, Lm, FP), jnp.float32),
        jax.ShapeDtypeStruct((n_radius, B, NB, Lm), jnp.float32),
        jax.ShapeDtypeStruct((T_STEPS + 1, B, FP), jnp.float32),
        jax.ShapeDtypeStruct((T_STEPS + 1, B, FP), jnp.float32),
        jax.ShapeDtypeStruct((T_STEPS, B, Lm), jnp.float32),
        jax.ShapeDtypeStruct((B, 1), jnp.float32),
    )
    out_specs = (
        pl.BlockSpec((n_radius + 1, MB, Lm, FP), lambda i: (0, i, 0, 0)),
        pl.BlockSpec((n_radius, MB, NB, Lm), lambda i: (0, i, 0, 0)),
        pl.BlockSpec((T_STEPS + 1, MB, FP), lambda i: (0, i, 0)),
        pl.BlockSpec((T_STEPS + 1, MB, FP), lambda i: (0, i, 0)),
        pl.BlockSpec((T_STEPS, MB, Lm), lambda i: (0, i, 0)),
        pl.BlockSpec((MB, 1), lambda i: (i, 0)),
    )

    afv, aawt, mfv, mfu, maw, pred = pl.pallas_call(
        _body,
        grid=(grid,),
        in_specs=tensor_specs + weight_specs,
        out_specs=out_specs,
        out_shape=out_shapes,
        compiler_params=pltpu.CompilerParams(
            dimension_semantics=("parallel",)),
    )(atom_list, bond_list, adt, bdt, atom_mask, *weight_args)

    aaw = jnp.swapaxes(aawt, 2, 3)
    return (afv, aaw[..., None], mfv, mfu, maw[..., None], pred)
